# baseline (speedup 1.0000x reference)
import functools

import jax
import jax.numpy as jnp
from jax import lax
from jax.experimental import pallas as pl
from jax.experimental.pallas import tpu as pltpu
from jax.experimental.pallas import tpu_sc as plsc

_B = 64
_N = 268
_NF = 268
_NH = 16
_NC = 2
_E = 7168
_NN = _N * _N
_EG = _E // 16
_ZG = _NN // 16
_NWORK = 32
_GPW = _B // _NWORK


_UNROLL = 4


def _sc_body(ei_hbm, ea_hbm, out_hbm, src_v, dst_v, ew_v, acc_v):
    c = lax.axis_index("c")
    s = lax.axis_index("s")
    wid = s * 2 + c

    zeros16 = jnp.zeros((16,), jnp.float32)

    tail_c = lax.iota(jnp.int32, 16) + (_N - 16)

    @plsc.parallel_loop(0, _N, unroll=4)
    def _(r):
        for k in range(_N // 16):
            acc_v[r, pl.ds(k * 16, 16)] = zeros16
        plsc.store_scatter(acc_v, [jnp.full((16,), r, jnp.int32), tail_c],
                           zeros16)

    for gi in range(_GPW):
        g = wid * _GPW + gi
        pltpu.sync_copy(ei_hbm.at[g, 0], src_v)
        pltpu.sync_copy(ei_hbm.at[g, 1], dst_v)
        pltpu.sync_copy(ea_hbm.at[g], ew_v)

        def edge_body(j, carry):
            for u in range(_UNROLL):
                off = pl.multiple_of(j * (16 * _UNROLL) + u * 16, 16)
                sv = src_v[pl.ds(off, 16)]
                dv = dst_v[pl.ds(off, 16)]
                ev = ew_v[pl.ds(off, 16)]
                plsc.addupdate_scatter(acc_v, [sv, dv], ev)
            return carry

        lax.fori_loop(0, _EG // _UNROLL, edge_body, 0)
        pltpu.sync_copy(acc_v, out_hbm.at[g])

        if gi != _GPW - 1:
            def rezero_body(j, carry):
                for u in range(_UNROLL):
                    off = pl.multiple_of(j * (16 * _UNROLL) + u * 16, 16)
                    sv = src_v[pl.ds(off, 16)]
                    dv = dst_v[pl.ds(off, 16)]
                    plsc.store_scatter(acc_v, [sv, dv], zeros16)
                return carry

            lax.fori_loop(0, _EG // _UNROLL, rezero_body, 0)


@functools.cache
def _sc_densify():
    return functools.partial(
        pl.kernel,
        mesh=plsc.VectorSubcoreMesh(core_axis_name="c", subcore_axis_name="s"),
        out_type=jax.ShapeDtypeStruct((_B, _N, _N), jnp.float32),
        compiler_params=pltpu.CompilerParams(needs_layout_passes=False),
        scratch_types=[
            pltpu.VMEM((_E,), jnp.int32),
            pltpu.VMEM((_E,), jnp.int32),
            pltpu.VMEM((_E,), jnp.float32),
            pltpu.VMEM((_N, _N), jnp.float32),
        ],
    )(_sc_body)


def _mm(a, b, dims):
    return lax.dot_general(a, b, (dims, ((), ())),
                           preferred_element_type=jnp.float32,
                           precision=lax.Precision.DEFAULT)


_TCG = 8


def _tc_body(wt_ref, x_ref, w1_ref, b1_ref, w4_ref, b4_ref,
             wc1_ref, bc1_ref, wc2_ref, bc2_ref, out_ref):
    G = range(_TCG)
    Wt = [wt_ref[i] for i in G]
    x = [x_ref[i] for i in G]

    dis = []
    for i in G:
        deg = jnp.sum(Wt[i], axis=1, keepdims=True)
        ok = deg > 0.0
        dis.append(jnp.where(ok, lax.rsqrt(jnp.where(ok, deg, 1.0)), 0.0))

    def prop(i, y):
        return -dis[i] * _mm(Wt[i], dis[i] * y, (((0,), (0,))))

    y = [_mm(x[i], w1_ref[...], (((1,), (0,)))) for i in G]
    p2 = [prop(i, y[i][:, 2 * _NH:]) for i in G]
    p1 = [prop(i, y[i][:, _NH:2 * _NH] + 2.0 * p2[i]) for i in G]
    h = [jax.nn.relu(y[i][:, :_NH] - y[i][:, 2 * _NH:] + p1[i] + b1_ref[...])
         for i in G]

    z = [_mm(h[i], w4_ref[...], (((1,), (0,)))) for i in G]
    q2 = [prop(i, z[i][:, 2 * _NC:]) for i in G]
    q1 = [prop(i, z[i][:, _NC:2 * _NC] + 2.0 * q2[i]) for i in G]
    out2 = [z[i][:, :_NC] - z[i][:, 2 * _NC:] + q1[i] + b4_ref[...] for i in G]

    t = [jax.nn.relu(_mm(out2[i], wc1_ref[...], (((0,), (0,)))) + bc1_ref[...])
         for i in G]
    o = [_mm(wc2_ref[...], t[i], (((0,), (1,)))) for i in G]
    for i in G:
        out_ref[i] = o[i] + bc2_ref[...]


def _tc_cheby(wt, x, W1, b1, W4, b4, Wc1, bc1, Wc2, bc2):
    full = lambda shp: pl.BlockSpec(shp, lambda g: (0,) * len(shp))
    return pl.pallas_call(
        _tc_body,
        grid=(_B // _TCG,),
        in_specs=[
            pl.BlockSpec((_TCG, _N, _N), lambda g: (g, 0, 0)),
            pl.BlockSpec((_TCG, _N, _NF), lambda g: (g, 0, 0)),
            full((_NF, 3 * _NH)),
            full((1, _NH)),
            full((_NH, 3 * _NC)),
            full((1, _NC)),
            full((_N, 50)),
            full((1, 50)),
            full((50, 1)),
            full((1, 1)),
        ],
        out_specs=pl.BlockSpec((_TCG, 1, _NC), lambda g: (g, 0, 0)),
        out_shape=jax.ShapeDtypeStruct((_B, 1, _NC), jnp.float32),
        compiler_params=pltpu.CompilerParams(
            dimension_semantics=("arbitrary",),
        ),
    )(wt, x, W1, b1, W4, b4, Wc1, bc1, Wc2, bc2)


@jax.jit
def kernel(x, edge_index, edge_attr, W1, b1, W4, b4, Wc1, bc1, Wc2, bc2):
    wt = _sc_densify()(edge_index, edge_attr)
    W1c = W1.transpose(1, 0, 2).reshape(_NF, 3 * _NH)
    W4c = W4.transpose(1, 0, 2).reshape(_NH, 3 * _NC)
    out = _tc_cheby(wt, x, W1c, b1.reshape(1, _NH), W4c, b4.reshape(1, _NC),
                    Wc1, bc1.reshape(1, 50), Wc2, bc2.reshape(1, 1))
    return out.reshape(_B, _NC)


# speedup vs baseline: 1.0061x; 1.0061x over previous
import functools

import jax
import jax.numpy as jnp
from jax import lax
from jax.experimental import pallas as pl
from jax.experimental.pallas import tpu as pltpu
from jax.experimental.pallas import tpu_sc as plsc

_B = 64
_N = 268
_NF = 268
_NH = 16
_NC = 2
_E = 7168
_NN = _N * _N
_EG = _E // 16
_ZG = _NN // 16
_NWORK = 32
_GPW = _B // _NWORK


_UNROLL = 4


_HB = _B // 2


def _make_sc_body(goff):
    def _sc_body(ei_hbm, ea_hbm, out_hbm, src_v, dst_v, ew_v, acc_v):
        c = lax.axis_index("c")
        s = lax.axis_index("s")
        wid = s * 2 + c
        g = goff + wid

        zeros16 = jnp.zeros((16,), jnp.float32)

        tail_c = lax.iota(jnp.int32, 16) + (_N - 16)

        @plsc.parallel_loop(0, _N, unroll=4)
        def _(r):
            for k in range(_N // 16):
                acc_v[r, pl.ds(k * 16, 16)] = zeros16
            plsc.store_scatter(acc_v, [jnp.full((16,), r, jnp.int32), tail_c],
                               zeros16)

        pltpu.sync_copy(ei_hbm.at[g, 0], src_v)
        pltpu.sync_copy(ei_hbm.at[g, 1], dst_v)
        pltpu.sync_copy(ea_hbm.at[g], ew_v)

        def edge_body(j, carry):
            for u in range(_UNROLL):
                off = pl.multiple_of(j * (16 * _UNROLL) + u * 16, 16)
                sv = src_v[pl.ds(off, 16)]
                dv = dst_v[pl.ds(off, 16)]
                ev = ew_v[pl.ds(off, 16)]
                plsc.addupdate_scatter(acc_v, [sv, dv], ev)
            return carry

        lax.fori_loop(0, _EG // _UNROLL, edge_body, 0)
        pltpu.sync_copy(acc_v, out_hbm.at[wid])

    return _sc_body


@functools.cache
def _sc_densify(goff):
    return functools.partial(
        pl.kernel,
        mesh=plsc.VectorSubcoreMesh(core_axis_name="c", subcore_axis_name="s"),
        out_type=jax.ShapeDtypeStruct((_HB, _N, _N), jnp.float32),
        compiler_params=pltpu.CompilerParams(needs_layout_passes=False),
        scratch_types=[
            pltpu.VMEM((_E,), jnp.int32),
            pltpu.VMEM((_E,), jnp.int32),
            pltpu.VMEM((_E,), jnp.float32),
            pltpu.VMEM((_N, _N), jnp.float32),
        ],
    )(_make_sc_body(goff))


def _mm(a, b, dims):
    return lax.dot_general(a, b, (dims, ((), ())),
                           preferred_element_type=jnp.float32,
                           precision=lax.Precision.DEFAULT)


_TCG = 8


def _tc_body(wt_ref, x_ref, w1_ref, b1_ref, w4_ref, b4_ref,
             wc1_ref, bc1_ref, wc2_ref, bc2_ref, out_ref):
    G = range(_TCG)
    Wt = [wt_ref[i] for i in G]
    x = [x_ref[i] for i in G]

    dis = []
    for i in G:
        deg = jnp.sum(Wt[i], axis=1, keepdims=True)
        ok = deg > 0.0
        dis.append(jnp.where(ok, lax.rsqrt(jnp.where(ok, deg, 1.0)), 0.0))

    def prop(i, y):
        return -dis[i] * _mm(Wt[i], dis[i] * y, (((0,), (0,))))

    y = [_mm(x[i], w1_ref[...], (((1,), (0,)))) for i in G]
    p2 = [prop(i, y[i][:, 2 * _NH:]) for i in G]
    p1 = [prop(i, y[i][:, _NH:2 * _NH] + 2.0 * p2[i]) for i in G]
    h = [jax.nn.relu(y[i][:, :_NH] - y[i][:, 2 * _NH:] + p1[i] + b1_ref[...])
         for i in G]

    z = [_mm(h[i], w4_ref[...], (((1,), (0,)))) for i in G]
    q2 = [prop(i, z[i][:, 2 * _NC:]) for i in G]
    q1 = [prop(i, z[i][:, _NC:2 * _NC] + 2.0 * q2[i]) for i in G]
    out2 = [z[i][:, :_NC] - z[i][:, 2 * _NC:] + q1[i] + b4_ref[...] for i in G]

    t = [jax.nn.relu(_mm(out2[i], wc1_ref[...], (((0,), (0,)))) + bc1_ref[...])
         for i in G]
    o = [_mm(wc2_ref[...], t[i], (((0,), (1,)))) for i in G]
    for i in G:
        out_ref[i] = o[i] + bc2_ref[...]


def _tc_cheby(xblk_off, wt, x, W1, b1, W4, b4, Wc1, bc1, Wc2, bc2):
    full = lambda shp: pl.BlockSpec(shp, lambda g: (0,) * len(shp))
    return pl.pallas_call(
        _tc_body,
        grid=(_HB // _TCG,),
        in_specs=[
            pl.BlockSpec((_TCG, _N, _N), lambda g: (g, 0, 0)),
            pl.BlockSpec((_TCG, _N, _NF), lambda g: (g + xblk_off, 0, 0)),
            full((_NF, 3 * _NH)),
            full((1, _NH)),
            full((_NH, 3 * _NC)),
            full((1, _NC)),
            full((_N, 50)),
            full((1, 50)),
            full((50, 1)),
            full((1, 1)),
        ],
        out_specs=pl.BlockSpec((_TCG, 1, _NC), lambda g: (g, 0, 0)),
        out_shape=jax.ShapeDtypeStruct((_HB, 1, _NC), jnp.float32),
        compiler_params=pltpu.CompilerParams(
            dimension_semantics=("arbitrary",),
        ),
    )(wt, x, W1, b1, W4, b4, Wc1, bc1, Wc2, bc2)


@jax.jit
def kernel(x, edge_index, edge_attr, W1, b1, W4, b4, Wc1, bc1, Wc2, bc2):
    wt_a = _sc_densify(0)(edge_index, edge_attr)
    wt_b = _sc_densify(_HB)(edge_index, edge_attr)
    W1c = W1.transpose(1, 0, 2).reshape(_NF, 3 * _NH)
    W4c = W4.transpose(1, 0, 2).reshape(_NH, 3 * _NC)
    args = (W1c, b1.reshape(1, _NH), W4c, b4.reshape(1, _NC),
            Wc1, bc1.reshape(1, 50), Wc2, bc2.reshape(1, 1))
    out_a = _tc_cheby(0, wt_a, x, *args)
    out_b = _tc_cheby(_HB // _TCG, wt_b, x, *args)
    return jnp.concatenate([out_a, out_b], axis=0).reshape(_B, _NC)
